# Initial kernel scaffold; baseline (speedup 1.0000x reference)
import jax
import jax.numpy as jnp
from jax.experimental import pallas as pl


def kernel(x, y):
    raise NotImplementedError("write your pallas kernel here")


# speedup vs baseline: 408.1489x; 408.1489x over previous
import functools

import jax
import jax.numpy as jnp
from jax.experimental import pallas as pl
from jax.experimental.pallas import tpu as pltpu

B = 4
N = 4096
TILE = 512
R = N // TILE


def _chamfer_kernel(x_ref, yt_ref, out_ref, colmin_ref, acc_ref):
    b = pl.program_id(0)
    r = pl.program_id(1)

    xt = x_ref[0]
    yt = yt_ref[0]

    cross = jax.lax.dot_general(
        xt, yt, (((1,), (0,)), ((), ())), preferred_element_type=jnp.float32
    )
    x2 = jnp.sum(xt * xt, axis=1, keepdims=True)
    y2 = jnp.sum(yt * yt, axis=0, keepdims=True)
    d2 = (x2 + y2) - (cross + cross)

    rowmin = jnp.min(d2, axis=1)
    colmin = jnp.min(d2, axis=0, keepdims=True)

    @pl.when(jnp.logical_and(b == 0, r == 0))
    def _init_acc():
        acc_ref[0] = 0.0

    @pl.when(r == 0)
    def _init_colmin():
        colmin_ref[...] = colmin

    @pl.when(r != 0)
    def _update_colmin():
        colmin_ref[...] = jnp.minimum(colmin_ref[...], colmin)

    acc_ref[0] += jnp.sum(rowmin)

    @pl.when(r == R - 1)
    def _fold_colmin():
        acc_ref[0] += jnp.sum(colmin_ref[...])

    @pl.when(jnp.logical_and(b == B - 1, r == R - 1))
    def _finish():
        out_ref[...] = jnp.full((1, 1), acc_ref[0] * (1.0 / (B * N)), jnp.float32)


@jax.jit
def kernel(x, y):
    yt = jnp.transpose(y, (0, 2, 1))
    out = pl.pallas_call(
        _chamfer_kernel,
        grid=(B, R),
        in_specs=[
            pl.BlockSpec((1, TILE, 3), lambda b, r: (b, r, 0)),
            pl.BlockSpec((1, 3, N), lambda b, r: (b, 0, 0)),
        ],
        out_specs=pl.BlockSpec((1, 1), lambda b, r: (0, 0)),
        out_shape=jax.ShapeDtypeStruct((1, 1), jnp.float32),
        scratch_shapes=[
            pltpu.VMEM((1, N), jnp.float32),
            pltpu.SMEM((1,), jnp.float32),
        ],
        compiler_params=pltpu.CompilerParams(
            dimension_semantics=("arbitrary", "arbitrary"),
        ),
    )(x, yt)
    return out[0, 0]


# speedup vs baseline: 456.9034x; 1.1195x over previous
import jax
import jax.numpy as jnp
from jax.experimental import pallas as pl
from jax.experimental.pallas import tpu as pltpu

B = 4
N = 4096
TILE = 1024
R = N // TILE


def _chamfer_kernel(xa_ref, ya_ref, out_ref, colmin_ref, acc_ref):
    b = pl.program_id(0)
    r = pl.program_id(1)

    xa = xa_ref[0]
    ya = ya_ref[0]

    d2 = jax.lax.dot_general(
        xa, ya, (((1,), (0,)), ((), ())), preferred_element_type=jnp.float32
    )

    rowmin = jnp.min(d2, axis=1)
    colmin = jnp.min(d2, axis=0, keepdims=True)

    @pl.when(jnp.logical_and(b == 0, r == 0))
    def _init_acc():
        acc_ref[0] = 0.0

    @pl.when(r == 0)
    def _init_colmin():
        colmin_ref[...] = colmin

    @pl.when(r != 0)
    def _update_colmin():
        colmin_ref[...] = jnp.minimum(colmin_ref[...], colmin)

    acc_ref[0] += jnp.sum(rowmin)

    @pl.when(r == R - 1)
    def _fold_colmin():
        acc_ref[0] += jnp.sum(colmin_ref[...])

    @pl.when(jnp.logical_and(b == B - 1, r == R - 1))
    def _finish():
        out_ref[...] = jnp.full((1, 1), acc_ref[0] * (1.0 / (B * N)), jnp.float32)


@jax.jit
def kernel(x, y):
    x2 = jnp.sum(x * x, axis=-1, keepdims=True)
    y2 = jnp.sum(y * y, axis=-1, keepdims=True)
    ones = jnp.ones_like(x2)
    xa = jnp.concatenate([x2, ones, -2.0 * x], axis=-1)
    ya = jnp.concatenate([ones, y2, y], axis=-1)
    yat = jnp.transpose(ya, (0, 2, 1))
    out = pl.pallas_call(
        _chamfer_kernel,
        grid=(B, R),
        in_specs=[
            pl.BlockSpec((1, TILE, 5), lambda b, r: (b, r, 0)),
            pl.BlockSpec((1, 5, N), lambda b, r: (b, 0, 0)),
        ],
        out_specs=pl.BlockSpec((1, 1), lambda b, r: (0, 0)),
        out_shape=jax.ShapeDtypeStruct((1, 1), jnp.float32),
        scratch_shapes=[
            pltpu.VMEM((1, N), jnp.float32),
            pltpu.SMEM((1,), jnp.float32),
        ],
        compiler_params=pltpu.CompilerParams(
            dimension_semantics=("arbitrary", "arbitrary"),
        ),
    )(xa, yat)
    return out[0, 0]


# speedup vs baseline: 528.7107x; 1.1572x over previous
import jax
import jax.numpy as jnp
from jax.experimental import pallas as pl
from jax.experimental.pallas import tpu as pltpu

B = 4
N = 4096
TILE = 1024
R = N // TILE


def _chamfer_kernel(x_ref, yt_ref, out_ref, colmin_ref, acc_ref):
    b = pl.program_id(0)
    r = pl.program_id(1)

    xt = x_ref[0]
    yt = yt_ref[0]

    cross = jax.lax.dot_general(
        xt, yt, (((1,), (0,)), ((), ())), preferred_element_type=jnp.float32
    )
    hx2 = 0.5 * jnp.sum(xt * xt, axis=1, keepdims=True)
    hy2 = 0.5 * jnp.sum(yt * yt, axis=0, keepdims=True)

    rowmin_half = jnp.min(hy2 - cross, axis=1)
    colmin_half = jnp.min(hx2 - cross, axis=0, keepdims=True)

    rowsum = jnp.sum(hx2) * 2.0 + jnp.sum(rowmin_half) * 2.0

    @pl.when(jnp.logical_and(b == 0, r == 0))
    def _init_acc():
        acc_ref[0] = 0.0

    @pl.when(r == 0)
    def _init_colmin():
        colmin_ref[...] = colmin_half

    @pl.when(r != 0)
    def _update_colmin():
        colmin_ref[...] = jnp.minimum(colmin_ref[...], colmin_half)

    acc_ref[0] += rowsum

    @pl.when(r == R - 1)
    def _fold_colmin():
        acc_ref[0] += jnp.sum(hy2) * 2.0 + jnp.sum(colmin_ref[...]) * 2.0

    @pl.when(jnp.logical_and(b == B - 1, r == R - 1))
    def _finish():
        out_ref[...] = jnp.full((1, 1), acc_ref[0] * (1.0 / (B * N)), jnp.float32)


@jax.jit
def kernel(x, y):
    yt = jnp.transpose(y, (0, 2, 1))
    out = pl.pallas_call(
        _chamfer_kernel,
        grid=(B, R),
        in_specs=[
            pl.BlockSpec((1, TILE, 3), lambda b, r: (b, r, 0)),
            pl.BlockSpec((1, 3, N), lambda b, r: (b, 0, 0)),
        ],
        out_specs=pl.BlockSpec((1, 1), lambda b, r: (0, 0)),
        out_shape=jax.ShapeDtypeStruct((1, 1), jnp.float32),
        scratch_shapes=[
            pltpu.VMEM((1, N), jnp.float32),
            pltpu.SMEM((1,), jnp.float32),
        ],
        compiler_params=pltpu.CompilerParams(
            dimension_semantics=("arbitrary", "arbitrary"),
        ),
    )(x, yt)
    return out[0, 0]


# speedup vs baseline: 556.0212x; 1.0517x over previous
import jax
import jax.numpy as jnp
from jax.experimental import pallas as pl
from jax.experimental.pallas import tpu as pltpu

B = 4
N = 4096
TILE = 2048
R = N // TILE


def _chamfer_kernel(x_ref, yt_ref, out_ref, colmin_ref, acc_ref):
    b = pl.program_id(0)
    r = pl.program_id(1)

    xt = x_ref[0]
    yt = yt_ref[0]

    cross = jax.lax.dot_general(
        xt, yt, (((1,), (0,)), ((), ())), preferred_element_type=jnp.float32
    )
    hx2 = 0.5 * jnp.sum(xt * xt, axis=1, keepdims=True)
    hy2 = 0.5 * jnp.sum(yt * yt, axis=0, keepdims=True)

    rowmin_half = jnp.min(hy2 - cross, axis=1)
    colmin_half = jnp.min(hx2 - cross, axis=0, keepdims=True)

    rowsum = jnp.sum(hx2) * 2.0 + jnp.sum(rowmin_half) * 2.0

    @pl.when(jnp.logical_and(b == 0, r == 0))
    def _init_acc():
        acc_ref[0] = 0.0

    @pl.when(r == 0)
    def _init_colmin():
        colmin_ref[...] = colmin_half

    @pl.when(r != 0)
    def _update_colmin():
        colmin_ref[...] = jnp.minimum(colmin_ref[...], colmin_half)

    acc_ref[0] += rowsum

    @pl.when(r == R - 1)
    def _fold_colmin():
        acc_ref[0] += jnp.sum(hy2) * 2.0 + jnp.sum(colmin_ref[...]) * 2.0

    @pl.when(jnp.logical_and(b == B - 1, r == R - 1))
    def _finish():
        out_ref[...] = jnp.full((1, 1), acc_ref[0] * (1.0 / (B * N)), jnp.float32)


@jax.jit
def kernel(x, y):
    yt = jnp.transpose(y, (0, 2, 1))
    out = pl.pallas_call(
        _chamfer_kernel,
        grid=(B, R),
        in_specs=[
            pl.BlockSpec((1, TILE, 3), lambda b, r: (b, r, 0)),
            pl.BlockSpec((1, 3, N), lambda b, r: (b, 0, 0)),
        ],
        out_specs=pl.BlockSpec((1, 1), lambda b, r: (0, 0)),
        out_shape=jax.ShapeDtypeStruct((1, 1), jnp.float32),
        scratch_shapes=[
            pltpu.VMEM((1, N), jnp.float32),
            pltpu.SMEM((1,), jnp.float32),
        ],
        compiler_params=pltpu.CompilerParams(
            dimension_semantics=("arbitrary", "arbitrary"),
        ),
    )(x, yt)
    return out[0, 0]
